# baseline (speedup 1.0000x reference)
import functools

import jax
import jax.numpy as jnp
from jax import lax
from jax.experimental import pallas as pl
from jax.experimental.pallas import tpu as pltpu
from jax.experimental.pallas import tpu_sc as plsc

H = 224
C = 96
CSC = 64
R = 16
NSTRIP = H // R
NCORES = 2
NSUB = 16
NW = NCORES * NSUB
CPW = CSC // NW
TPW = CPW * NSTRIP
NCHUNK = H // 16
PLANE = H * H
STRIPW = R * H
NTI = (C * 9) // 8
NTJ = PLANE // 128
STAGE = 32
BUFT = 36
TJMAX = NTJ - STAGE


def _build_sc_call():
    mesh = plsc.VectorSubcoreMesh(core_axis_name="c", subcore_axis_name="s")

    @functools.partial(
        pl.kernel,
        out_type=jax.ShapeDtypeStruct((CSC, H, H), jnp.float32),
        mesh=mesh,
        compiler_params=pltpu.CompilerParams(
            use_tc_tiling_on_sc=False, needs_layout_passes=False),
        scratch_types=[
            pltpu.VMEM((2, 9, BUFT, 128), jnp.float32),
            pltpu.VMEM((2, R, H), jnp.float32),
            pltpu.SemaphoreType.DMA,
            pltpu.SemaphoreType.DMA,
            pltpu.SemaphoreType.DMA,
            pltpu.SemaphoreType.DMA,
        ],
    )
    def col2im_sc(x_hbm, out_hbm, ibuf, obuf, isem0, isem1, osem0, osem1):
        wid = lax.axis_index("s") * NCORES + lax.axis_index("c")
        base_t = wid * TPW
        isem = (isem0, isem1)
        osem = (osem0, osem1)
        zeros16 = jnp.zeros((16,), jnp.float32)
        lane = lax.iota(jnp.int32, 16)
        lane_f = lane.astype(jnp.float32)
        mask_lo = jnp.minimum(lane_f, 1.0)
        mask_hi = jnp.minimum(15.0 - lane_f, 1.0)

        def zpad(i, carry):
            for bb in range(2):
                for p in range(9):
                    for tt in (0, 1, BUFT - 2, BUFT - 1):
                        ibuf[bb, p, tt, pl.ds(i * 16, 16)] = zeros16
            return carry

        lax.fori_loop(0, 128 // 16, zpad, 0)

        def split(t):
            c = t // NSTRIP
            s = t - c * NSTRIP
            return c, s

        def in_copies(t, bb):
            c, s = split(t)
            b = s * STRIPW
            tja = jnp.clip(b - 256, 0, TJMAX * 128) // 128
            cps = []
            for p in range(9):
                q = c * 9 + p
                ti = q // 8
                ii = q - ti * 8
                cps.append(pltpu.make_async_copy(
                    x_hbm.at[ti, pl.ds(tja, STAGE), ii, :],
                    ibuf.at[bb, p, pl.ds(2, STAGE), :],
                    isem[bb]))
            return cps

        def out_copy(t, bb):
            c, s = split(t)
            return pltpu.make_async_copy(
                obuf.at[bb], out_hbm.at[c, pl.ds(s * R, R), :], osem[bb])

        def issue_in(t, bb):
            for cp in in_copies(t, bb):
                cp.start()

        def wait_in(t, bb):
            for cp in in_copies(t, bb):
                cp.wait()

        def compute(t, bb):
            _, s = split(t)
            b = s * STRIPW
            tja = jnp.clip(b - 256, 0, TJMAX * 128) // 128
            A = 256 + b - tja * 128
            ashift = A // 128

            def gbody(g, cc):
                sg = ashift + 7 * g
                rowvec = [A + g * 896 + j * 224 + lane for j in range(4)]
                for j in range(4):
                    r = 4 * g + j
                    for ch in range(NCHUNK):
                        zo = j * 224 + ch * 16
                        accs = []
                        for kw in (0, 1, 2):
                            acc = None
                            for kh in (0, 1, 2):
                                p = kh * 3 + kw
                                d = (1 - kh) * 224 + (1 - kw)
                                if kw == 1:
                                    off = zo + d
                                    v = ibuf[bb, p, sg + off // 128,
                                             pl.ds(off % 128, 16)]
                                else:
                                    idx = rowvec[j] + (ch * 16 + d)
                                    t_idx = lax.shift_right_logical(idx, 7)
                                    c_idx = lax.bitwise_and(idx, 127)
                                    v = plsc.load_gather(
                                        ibuf.at[bb, p], [t_idx, c_idx])
                                acc = v if acc is None else acc + v
                            accs.append(acc)
                        a0, a1, a2 = accs
                        if ch == NCHUNK - 1:
                            a0 = a0 * mask_hi
                        if ch == 0:
                            a2 = a2 * mask_lo
                        obuf[bb, r, pl.ds(ch * 16, 16)] = a0 + a1 + a2
                return cc

            lax.fori_loop(0, 4, gbody, 0)

        issue_in(base_t, 0)

        def pair(g, carry):
            t0 = base_t + 2 * g
            issue_in(t0 + 1, 1)
            wait_in(t0, 0)

            @pl.when(g > 0)
            def _():
                out_copy(t0, 0).wait()

            compute(t0, 0)
            out_copy(t0, 0).start()

            @pl.when(g < TPW // 2 - 1)
            def _():
                issue_in(t0 + 2, 0)

            wait_in(t0 + 1, 1)

            @pl.when(g > 0)
            def _():
                out_copy(t0 + 1, 1).wait()

            compute(t0 + 1, 1)
            out_copy(t0 + 1, 1).start()
            return carry

        lax.fori_loop(0, TPW // 2, pair, 0)
        out_copy(base_t, 0).wait()
        out_copy(base_t, 1).wait()

    return col2im_sc


_COL2IM_SC = _build_sc_call()

CTC = C - CSC


def _tc_body(x_ref, o_ref):
    planes = x_ref[0].reshape(9, H, H)
    acc = jnp.zeros((H, H), jnp.float32)
    for kh in range(3):
        for kw in range(3):
            p = kh * 3 + kw
            r0, r1 = max(0, kh - 1), H + min(0, kh - 1)
            c0, c1 = max(0, kw - 1), H + min(0, kw - 1)
            term = lax.slice(planes[p],
                             (r0 + 1 - kh, c0 + 1 - kw),
                             (r1 + 1 - kh, c1 + 1 - kw))
            acc = acc + lax.pad(term, jnp.float32(0),
                                ((r0, H - r1, 0), (c0, H - c1, 0)))
    o_ref[0] = acc


def _build_tc_call():
    return pl.pallas_call(
        _tc_body,
        grid=(CTC,),
        in_specs=[pl.BlockSpec((1, 9, PLANE), lambda i: (CSC + i, 0, 0))],
        out_specs=pl.BlockSpec((1, H, H), lambda i: (i, 0, 0)),
        out_shape=jax.ShapeDtypeStruct((CTC, H, H), jnp.float32),
        compiler_params=pltpu.CompilerParams(
            dimension_semantics=("arbitrary",)),
    )


_COL2IM_TC = _build_tc_call()


def kernel(x, output_size, kernel_size, dilation, padding, stride):
    x4t = lax.transpose(x.reshape(NTI, 8, NTJ, 128), (0, 2, 1, 3))
    out_sc = _COL2IM_SC(x4t)
    x3 = x.reshape(C * 9, H, H).reshape(C, 9, PLANE)
    out_tc = _COL2IM_TC(x3)
    out = jnp.concatenate([out_sc, out_tc], axis=0)
    return out.reshape(1, C, H, H)


# speedup vs baseline: 1.7328x; 1.7328x over previous
import functools

import jax
import jax.numpy as jnp
from jax import lax
from jax.experimental import pallas as pl
from jax.experimental.pallas import tpu as pltpu
from jax.experimental.pallas import tpu_sc as plsc

H = 224
C = 96
R = 16
NSTRIP = H // R
NCORES = 2
NSUB = 16
NW = NCORES * NSUB
CPW = C // NW
TPW = CPW * NSTRIP
NCHUNK = H // 16
PLANE = H * H
STRIPW = R * H
NTI = (C * 9) // 8
NTJ = PLANE // 128
STAGE = 32
BUFT = 36
NBUF = 3
SLOTT = 9 * 34 + 2
TJMAX = NTJ - STAGE


def _build_sc_call():
    mesh = plsc.VectorSubcoreMesh(core_axis_name="c", subcore_axis_name="s")

    @functools.partial(
        pl.kernel,
        out_type=jax.ShapeDtypeStruct((C, H, H), jnp.float32),
        mesh=mesh,
        compiler_params=pltpu.CompilerParams(
            use_tc_tiling_on_sc=False, needs_layout_passes=False),
        scratch_types=[
            pltpu.VMEM((NBUF, SLOTT, 128), jnp.float32),
            pltpu.VMEM((NBUF, R, H), jnp.float32),
            pltpu.SemaphoreType.DMA,
            pltpu.SemaphoreType.DMA,
            pltpu.SemaphoreType.DMA,
            pltpu.SemaphoreType.DMA,
            pltpu.SemaphoreType.DMA,
            pltpu.SemaphoreType.DMA,
        ],
    )
    def col2im_sc(x_hbm, out_hbm, ibuf, obuf,
                  isem0, isem1, isem2, osem0, osem1, osem2):
        wid = lax.axis_index("s") * NCORES + lax.axis_index("c")
        base_t = wid * TPW
        isem = (isem0, isem1, isem2)
        osem = (osem0, osem1, osem2)
        zeros16 = jnp.zeros((16,), jnp.float32)
        lane = lax.iota(jnp.int32, 16)
        lane_f = lane.astype(jnp.float32)
        mask_lo = jnp.minimum(lane_f, 1.0)
        mask_hi = jnp.minimum(15.0 - lane_f, 1.0)

        def zpad(i, carry):
            for bb in range(NBUF):
                for p in range(10):
                    for tt in (34 * p, 34 * p + 1):
                        ibuf[bb, tt, pl.ds(i * 16, 16)] = zeros16
            return carry

        lax.fori_loop(0, 128 // 16, zpad, 0)

        def split(t):
            c = t // NSTRIP
            s = t - c * NSTRIP
            return c, s

        def in_copies(t, bb):
            c, s = split(t)
            b = s * STRIPW
            tja = jnp.clip(b - 256, 0, TJMAX * 128) // 128
            cps = []
            for p in range(9):
                q = c * 9 + p
                ti = q // 8
                ii = q - ti * 8
                cps.append(pltpu.make_async_copy(
                    x_hbm.at[ti, pl.ds(tja, STAGE), ii, :],
                    ibuf.at[bb, pl.ds(34 * p + 2, STAGE), :],
                    isem[bb]))
            return cps

        def out_copy(t, bb):
            c, s = split(t)
            return pltpu.make_async_copy(
                obuf.at[bb], out_hbm.at[c, pl.ds(s * R, R), :], osem[bb])

        def issue_in(t, bb):
            for cp in in_copies(t, bb):
                cp.start()

        def wait_in(t, bb):
            for cp in in_copies(t, bb):
                cp.wait()

        def compute(t, bb):
            _, s = split(t)
            b = s * STRIPW
            tja = jnp.clip(b - 256, 0, TJMAX * 128) // 128
            A = 256 + b - tja * 128
            ashift = A // 128

            def gbody(g, cc):
                sg = ashift + 7 * g
                rowvec = [A + g * 896 + j * 224 + lane for j in range(4)]
                for j in range(4):
                    r = 4 * g + j
                    for ch in range(NCHUNK):
                        zo = j * 224 + ch * 16
                        accs = []
                        for kw in (0, 1, 2):
                            acc = None
                            for kh in (0, 1, 2):
                                p = kh * 3 + kw
                                d = (1 - kh) * 224 + (1 - kw)
                                if kw == 1:
                                    off = zo + d
                                    v = ibuf[bb, 34 * p + sg + off // 128,
                                             pl.ds(off % 128, 16)]
                                else:
                                    idx = rowvec[j] + (ch * 16 + d)
                                    t_idx = lax.shift_right_logical(idx, 7)
                                    c_idx = lax.bitwise_and(idx, 127)
                                    v = plsc.load_gather(
                                        ibuf.at[bb], [34 * p + t_idx, c_idx])
                                acc = v if acc is None else acc + v
                            accs.append(acc)
                        a0, a1, a2 = accs
                        if ch == NCHUNK - 1:
                            a0 = a0 * mask_hi
                        if ch == 0:
                            a2 = a2 * mask_lo
                        obuf[bb, r, pl.ds(ch * 16, 16)] = a0 + a1 + a2
                return cc

            lax.fori_loop(0, 4, gbody, 0)

        issue_in(base_t, 0)
        issue_in(base_t + 1, 1)

        def triple(g, carry):
            t0 = base_t + 3 * g
            for k in range(3):
                t = t0 + k
                nxt = (k + 2) % 3
                @pl.when(t + 2 < base_t + TPW)
                def _():
                    issue_in(t + 2, nxt)
                wait_in(t, k)

                @pl.when(g > 0)
                def _():
                    out_copy(t, k).wait()

                compute(t, k)
                out_copy(t, k).start()
            return carry

        lax.fori_loop(0, TPW // 3, triple, 0)
        out_copy(base_t, 0).wait()
        out_copy(base_t, 1).wait()
        out_copy(base_t, 2).wait()

    return col2im_sc


_COL2IM_SC = _build_sc_call()


def kernel(x, output_size, kernel_size, dilation, padding, stride):
    x4t = lax.transpose(x.reshape(NTI, 8, NTJ, 128), (0, 2, 1, 3))
    out = _COL2IM_SC(x4t)
    return out.reshape(1, C, H, H)
